# baseline (speedup 1.0000x reference)
import functools

import jax
import jax.numpy as jnp
from jax.experimental import pallas as pl
from jax.experimental.pallas import tpu as pltpu

_DEPTH = 4


def _layer_kernel(e_ref, a_hbm, o_ref, buf, sem, *, n: int, bm: int, d: int, dp: int):
    nblk = n // bm

    def copy(b):
        return pltpu.make_async_copy(
            a_hbm.at[pl.ds(b * bm, bm), :],
            buf.at[b % _DEPTH],
            sem.at[b % _DEPTH],
        )

    for b in range(_DEPTH):
        copy(b).start()
    for b in range(nblk):
        copy(b).wait()
        acc = jnp.dot(buf[b % _DEPTH], e_ref[...], preferred_element_type=jnp.float32)
        if b + _DEPTH < nblk:
            copy(b + _DEPTH).start()
        e_rows = e_ref[pl.ds(b * bm, bm), :]
        scale = 1.0 / (acc[:, d] + 1.0)
        res = (e_rows + acc) * scale[:, None]
        col = jax.lax.broadcasted_iota(jnp.int32, (bm, dp), 1)
        o_ref[pl.ds(b * bm, bm), :] = jnp.where(col == d, 1.0, res)


def kernel(embedding, bottom_to_top_paths):
    n, d = embedding.shape
    p = bottom_to_top_paths.shape[0]
    dp = 128
    bm = 512

    e = jnp.concatenate(
        [
            embedding,
            jnp.ones((n, 1), jnp.float32),
            jnp.zeros((n, dp - d - 1), jnp.float32),
        ],
        axis=1,
    )

    layer = pl.pallas_call(
        functools.partial(_layer_kernel, n=n, bm=bm, d=d, dp=dp),
        grid=(1,),
        in_specs=[
            pl.BlockSpec((n, dp), lambda i: (0, 0)),
            pl.BlockSpec(memory_space=pltpu.MemorySpace.HBM),
        ],
        out_specs=pl.BlockSpec((n, dp), lambda i: (0, 0)),
        out_shape=jax.ShapeDtypeStruct((n, dp), jnp.float32),
        scratch_shapes=[
            pltpu.VMEM((_DEPTH, bm, n), jnp.float32),
            pltpu.SemaphoreType.DMA((_DEPTH,)),
        ],
    )

    for pi in range(p):
        e = layer(e, bottom_to_top_paths[pi])
    return e[:, :d]


# speedup vs baseline: 2.6434x; 2.6434x over previous
import functools

import jax
import jax.numpy as jnp
from jax.experimental import pallas as pl
from jax.experimental.pallas import tpu as pltpu


def _layer_kernel(e_ref, a_ref, o_ref, *, bm: int, d: int, dp: int):
    i = pl.program_id(0)
    acc = jnp.dot(a_ref[0], e_ref[...], preferred_element_type=jnp.float32)
    e_rows = e_ref[pl.ds(i * bm, bm), :]
    scale = 1.0 / (acc[:, d] + 1.0)
    res = (e_rows + acc) * scale[:, None]
    col = jax.lax.broadcasted_iota(jnp.int32, (bm, dp), 1)
    o_ref[...] = jnp.where(col == d, 1.0, res)


def kernel(embedding, bottom_to_top_paths):
    n, d = embedding.shape
    p = bottom_to_top_paths.shape[0]
    dp = 128
    bm = 512

    e = jnp.concatenate(
        [
            embedding,
            jnp.ones((n, 1), jnp.float32),
            jnp.zeros((n, dp - d - 1), jnp.float32),
        ],
        axis=1,
    )

    def make_layer(pi):
        return pl.pallas_call(
            functools.partial(_layer_kernel, bm=bm, d=d, dp=dp),
            grid=(n // bm,),
            in_specs=[
                pl.BlockSpec((n, dp), lambda i: (0, 0)),
                pl.BlockSpec((1, bm, n), lambda i, pi=pi: (pi, i, 0)),
            ],
            out_specs=pl.BlockSpec((bm, dp), lambda i: (i, 0)),
            out_shape=jax.ShapeDtypeStruct((n, dp), jnp.float32),
            compiler_params=pltpu.CompilerParams(
                dimension_semantics=("arbitrary",),
            ),
        )

    for pi in range(p):
        e = make_layer(pi)(e, bottom_to_top_paths)
    return e[:, :d]
